# baseline (speedup 1.0000x reference)
import functools

import jax
import jax.numpy as jnp
from jax import lax
from jax.experimental import pallas as pl
from jax.experimental.pallas import tpu as pltpu
from jax.experimental.pallas import tpu_sc as plsc

B = 16384
D = 32
NC = 2
NS = 16
NW = NC * NS
BPW = B // NW
CH = 128
NCH = BPW // CH
LANES = 16

_mesh = plsc.VectorSubcoreMesh(core_axis_name="c", subcore_axis_name="s")


@functools.partial(
    pl.kernel,
    out_type=(
        jax.ShapeDtypeStruct((NW, LANES), jnp.float32),
        jax.ShapeDtypeStruct((B,), jnp.float32),
    ),
    mesh=_mesh,
    scratch_types=[
        pltpu.VMEM((NCH, CH), jnp.int32),
        pltpu.VMEM((NCH, CH), jnp.int32),
        pltpu.VMEM((BPW, D), jnp.float32),
        pltpu.VMEM((BPW, D), jnp.float32),
        pltpu.VMEM((BPW,), jnp.float32),
        pltpu.VMEM((BPW,), jnp.float32),
        pltpu.VMEM((BPW,), jnp.float32),
        pltpu.VMEM((LANES,), jnp.float32),
        pltpu.SemaphoreType.DMA,
    ],
    compiler_params=pltpu.CompilerParams(use_tc_tiling_on_sc=False),
)
def _gather_dot(uidx_hbm, iidx_hbm, uemb_hbm, iemb_hbm, ubias_hbm, ibias_hbm,
                part_out, bsum_out,
                uidx_v, iidx_v, urows, irows, ub_v, ib_v, bs_v, acc_v, sem):
    c = lax.axis_index("c")
    s = lax.axis_index("s")
    wid = s * NC + c
    base = wid * BPW

    pltpu.sync_copy(uidx_hbm.at[pl.ds(wid * NCH, NCH), :], uidx_v)
    pltpu.sync_copy(iidx_hbm.at[pl.ds(wid * NCH, NCH), :], iidx_v)

    copies = []
    for j in range(NCH):
        copies.append(pltpu.async_copy(
            uemb_hbm.at[uidx_v.at[j]], urows.at[pl.ds(j * CH, CH), :], sem))
        copies.append(pltpu.async_copy(
            iemb_hbm.at[iidx_v.at[j]], irows.at[pl.ds(j * CH, CH), :], sem))
        copies.append(pltpu.async_copy(
            ubias_hbm.at[uidx_v.at[j]], ub_v.at[pl.ds(j * CH, CH)], sem))
        copies.append(pltpu.async_copy(
            ibias_hbm.at[iidx_v.at[j]], ib_v.at[pl.ds(j * CH, CH)], sem))
    for cp in copies:
        cp.wait()

    def bias_body(j, carry):
        sl = pl.ds(j * LANES, LANES)
        bs_v[sl] = ub_v[sl] + ib_v[sl]
        return carry

    lax.fori_loop(0, BPW // LANES, bias_body, 0)
    pltpu.sync_copy(bs_v, bsum_out.at[pl.ds(base, BPW)])

    def dot_body(i, acc):
        a = urows[i, pl.ds(0, LANES)] * irows[i, pl.ds(0, LANES)]
        b = urows[i, pl.ds(LANES, LANES)] * irows[i, pl.ds(LANES, LANES)]
        return acc + a + b

    acc = lax.fori_loop(0, BPW, dot_body, jnp.zeros((LANES,), jnp.float32))
    acc_v[pl.ds(0, LANES)] = acc
    pltpu.sync_copy(acc_v, part_out.at[wid])


@functools.partial(
    pl.kernel,
    out_type=jax.ShapeDtypeStruct((B,), jnp.float32),
    mesh=_mesh,
    scratch_types=[
        pltpu.VMEM((NW, LANES), jnp.float32),
        pltpu.VMEM((BPW,), jnp.float32),
        pltpu.VMEM((BPW,), jnp.float32),
    ],
    compiler_params=pltpu.CompilerParams(
        use_tc_tiling_on_sc=False, needs_layout_passes=False),
)
def _finish(part_hbm, bsum_hbm, out_hbm, pv, bs_v, o_v):
    c = lax.axis_index("c")
    s = lax.axis_index("s")
    wid = s * NC + c
    base = wid * BPW

    pltpu.sync_copy(part_hbm, pv)
    pltpu.sync_copy(bsum_hbm.at[pl.ds(base, BPW)], bs_v)

    def sum_body(j, acc):
        return acc + pv[j, pl.ds(0, LANES)]

    acc = lax.fori_loop(0, NW, sum_body, jnp.zeros((LANES,), jnp.float32))
    total = jnp.sum(acc)
    s_vec = jnp.full((LANES,), total, dtype=jnp.float32)

    def out_body(j, carry):
        sl = pl.ds(j * LANES, LANES)
        x = s_vec + bs_v[sl]
        o_v[sl] = 1.0 / (1.0 + jnp.exp(-x))
        return carry

    lax.fori_loop(0, BPW // LANES, out_body, 0)
    pltpu.sync_copy(o_v, out_hbm.at[pl.ds(base, BPW)])


def kernel(inputs, user_embedding, user_bias, item_embedding, item_bias):
    idx = inputs.astype(jnp.int32)
    uidx = idx[:, 0].reshape(NW * NCH, CH)
    iidx = idx[:, 1].reshape(NW * NCH, CH)
    ub = user_bias.reshape(-1)
    ib = item_bias.reshape(-1)
    part, bsum = _gather_dot(uidx, iidx, user_embedding, item_embedding, ub, ib)
    out = _finish(part, bsum)
    return out.reshape(B, 1)


# speedup vs baseline: 1.0021x; 1.0021x over previous
import functools

import jax
import jax.numpy as jnp
from jax import lax
from jax.experimental import pallas as pl
from jax.experimental.pallas import tpu as pltpu
from jax.experimental.pallas import tpu_sc as plsc

B = 16384
D = 32
NC = 2
NS = 16
NW = NC * NS
BPW = B // NW
CH = 128
NCH = BPW // CH
LANES = 16

_mesh = plsc.VectorSubcoreMesh(core_axis_name="c", subcore_axis_name="s")


@functools.partial(
    pl.kernel,
    out_type=(
        jax.ShapeDtypeStruct((NW, LANES), jnp.float32),
        jax.ShapeDtypeStruct((B,), jnp.float32),
    ),
    mesh=_mesh,
    scratch_types=[
        pltpu.VMEM((NCH, CH), jnp.int32),
        pltpu.VMEM((NCH, CH), jnp.int32),
        pltpu.VMEM((BPW, D), jnp.float32),
        pltpu.VMEM((BPW, D), jnp.float32),
        pltpu.VMEM((BPW,), jnp.float32),
        pltpu.VMEM((BPW,), jnp.float32),
        pltpu.VMEM((BPW,), jnp.float32),
        pltpu.VMEM((LANES,), jnp.float32),
        pltpu.SemaphoreType.DMA,
    ],
    compiler_params=pltpu.CompilerParams(use_tc_tiling_on_sc=False),
)
def _gather_dot(uidx_hbm, iidx_hbm, uemb_hbm, iemb_hbm, ubias_hbm, ibias_hbm,
                part_out, bsum_out,
                uidx_v, iidx_v, urows, irows, ub_v, ib_v, bs_v, acc_v, sem):
    c = lax.axis_index("c")
    s = lax.axis_index("s")
    wid = s * NC + c
    base = wid * BPW

    pltpu.sync_copy(uidx_hbm.at[pl.ds(wid * NCH, NCH), :], uidx_v)
    pltpu.sync_copy(iidx_hbm.at[pl.ds(wid * NCH, NCH), :], iidx_v)

    copies = []
    for j in range(NCH):
        copies.append(pltpu.async_copy(
            uemb_hbm.at[uidx_v.at[j]], urows.at[pl.ds(j * CH, CH), :], sem))
        copies.append(pltpu.async_copy(
            iemb_hbm.at[iidx_v.at[j]], irows.at[pl.ds(j * CH, CH), :], sem))
        copies.append(pltpu.async_copy(
            ubias_hbm.at[uidx_v.at[j]], ub_v.at[pl.ds(j * CH, CH)], sem))
        copies.append(pltpu.async_copy(
            ibias_hbm.at[iidx_v.at[j]], ib_v.at[pl.ds(j * CH, CH)], sem))
    for cp in copies:
        cp.wait()

    def bias_body(j, carry):
        sl = pl.ds(j * LANES, LANES)
        bs_v[sl] = ub_v[sl] + ib_v[sl]
        return carry

    lax.fori_loop(0, BPW // LANES, bias_body, 0)
    pltpu.sync_copy(bs_v, bsum_out.at[pl.ds(base, BPW)])

    def dot_body(i, acc):
        a = urows[i, pl.ds(0, LANES)] * irows[i, pl.ds(0, LANES)]
        b = urows[i, pl.ds(LANES, LANES)] * irows[i, pl.ds(LANES, LANES)]
        return acc + a + b

    acc = lax.fori_loop(0, BPW, dot_body, jnp.zeros((LANES,), jnp.float32))
    acc_v[pl.ds(0, LANES)] = acc
    pltpu.sync_copy(acc_v, part_out.at[wid])


def _tc_finish_body(part_ref, bs_ref, o_ref):
    total = jnp.sum(part_ref[...])
    o_ref[...] = 1.0 / (1.0 + jnp.exp(-(bs_ref[...] + total)))


_tc_finish = pl.pallas_call(
    _tc_finish_body,
    out_shape=jax.ShapeDtypeStruct((128, 128), jnp.float32),
)


def kernel(inputs, user_embedding, user_bias, item_embedding, item_bias):
    idx = inputs.astype(jnp.int32)
    uidx = idx[:, 0].reshape(NW * NCH, CH)
    iidx = idx[:, 1].reshape(NW * NCH, CH)
    ub = user_bias.reshape(-1)
    ib = item_bias.reshape(-1)
    part, bsum = _gather_dot(uidx, iidx, user_embedding, item_embedding, ub, ib)
    out = _tc_finish(part, bsum.reshape(128, 128))
    return out.reshape(B, 1)
